# baseline (speedup 1.0000x reference)
import functools

import jax
import jax.numpy as jnp
from jax import lax
from jax.experimental import pallas as pl
from jax.experimental.pallas import tpu as pltpu, tpu_sc as plsc

WORD_COUNT = 1000000
NUM_BUCKETS = 100000
EMBED = 64
BATCH = 4096
SEQ = 200

NC = 2
NS = 16
L = 16
NW = NC * NS

N_TOK = BATCH * SEQ
TOK_PER_W = N_TOK // NW
CHUNK = 128
N_CHUNKS = TOK_PER_W // CHUNK
BLK = 8
N_BLK = WORD_COUNT // BLK


def _sc_body(tok_hbm, htb_hbm, pb_hbm, w_hbm, out_hbm,
             tok_v, wp_v, hidx_v, pidx_v, hmeta_v, pmeta_v,
             idx0_v, idx1_v, p0_v, p1_v, w0_v, w1_v, out_v, sem0, sem1):
    wid = lax.axis_index("s") * NC + lax.axis_index("c")
    tile_base = wid * TOK_PER_W
    lane = lax.iota(jnp.int32, L)

    def chunk_body(c, carry):
        base = tile_base + c * CHUNK
        pltpu.sync_copy(tok_hbm.at[pl.ds(base, CHUNK)], tok_v)

        def idx_body(g, _):
            s = g * L
            wv = tok_v[pl.ds(s, L)]
            t = wv + 3
            wp = jnp.where(t >= WORD_COUNT, t - WORD_COUNT, t)
            wp_v[pl.ds(s, L)] = wp
            hidx_v[pl.ds(s, L)] = lax.shift_right_logical(wv, 3)
            pidx_v[pl.ds(s, L)] = lax.shift_right_logical(wp, 3)
            return 0

        lax.fori_loop(0, CHUNK // L, idx_body, 0)

        cph = pltpu.async_copy(htb_hbm.at[hidx_v], hmeta_v, sem0)
        cpp = pltpu.async_copy(pb_hbm.at[pidx_v], pmeta_v, sem1)
        cph.wait()
        cpp.wait()

        def meta_body(g, _):
            s = g * L
            rows = s + lane
            wv = tok_v[pl.ds(s, L)]
            nz = wv != 0
            hc = (wv & 7) * 2
            b0 = plsc.load_gather(hmeta_v, [rows, hc])
            b1 = plsc.load_gather(hmeta_v, [rows, hc + 1])
            idx0_v[pl.ds(s, L)] = jnp.where(nz, b0, 0)
            idx1_v[pl.ds(s, L)] = jnp.where(nz, b1, 0)
            wpv = wp_v[pl.ds(s, L)]
            pc = (wpv & 7) * 2
            p0 = plsc.load_gather(pmeta_v, [rows, pc])
            p1 = plsc.load_gather(pmeta_v, [rows, pc + 1])
            p0_v[pl.ds(s, L)] = p0
            p1_v[pl.ds(s, L)] = p1
            plsc.store_scatter(out_v, [rows, jnp.full((L,), EMBED, jnp.int32)], p0)
            plsc.store_scatter(out_v, [rows, jnp.full((L,), EMBED + 1, jnp.int32)], p1)
            return 0

        lax.fori_loop(0, CHUNK // L, meta_body, 0)

        cp0 = pltpu.async_copy(w_hbm.at[idx0_v], w0_v, sem0)
        cp1 = pltpu.async_copy(w_hbm.at[idx1_v], w1_v, sem1)
        cp0.wait()
        cp1.wait()

        def tok_body(i, _):
            p0 = plsc.load_gather(p0_v, [jnp.full((L,), i, jnp.int32)])
            p1 = plsc.load_gather(p1_v, [jnp.full((L,), i, jnp.int32)])
            for k in range(EMBED // L):
                a = w0_v[i, pl.ds(k * L, L)]
                b = w1_v[i, pl.ds(k * L, L)]
                out_v[i, pl.ds(k * L, L)] = a * p0 + b * p1
            return 0

        lax.fori_loop(0, CHUNK, tok_body, 0)

        pltpu.sync_copy(out_v, out_hbm.at[pl.ds(base, CHUNK)])
        return carry

    lax.fori_loop(0, N_CHUNKS, chunk_body, 0)


def kernel(input, hash_tables, p, W):
    tok = input.reshape(N_TOK)
    htb = hash_tables.reshape(N_BLK, BLK * 2)
    pb = p.reshape(N_BLK, BLK * 2)

    mesh = plsc.VectorSubcoreMesh(
        core_axis_name="c", subcore_axis_name="s",
        num_cores=NC, num_subcores=NS)
    run = pl.kernel(
        _sc_body,
        out_type=jax.ShapeDtypeStruct((N_TOK, EMBED + 2), jnp.float32),
        mesh=mesh,
        compiler_params=pltpu.CompilerParams(
            needs_layout_passes=False, use_tc_tiling_on_sc=False),
        scratch_types=[
            pltpu.VMEM((CHUNK,), jnp.int32),
            pltpu.VMEM((CHUNK,), jnp.int32),
            pltpu.VMEM((CHUNK,), jnp.int32),
            pltpu.VMEM((CHUNK,), jnp.int32),
            pltpu.VMEM((CHUNK, BLK * 2), jnp.int32),
            pltpu.VMEM((CHUNK, BLK * 2), jnp.float32),
            pltpu.VMEM((CHUNK,), jnp.int32),
            pltpu.VMEM((CHUNK,), jnp.int32),
            pltpu.VMEM((CHUNK,), jnp.float32),
            pltpu.VMEM((CHUNK,), jnp.float32),
            pltpu.VMEM((CHUNK, EMBED), jnp.float32),
            pltpu.VMEM((CHUNK, EMBED), jnp.float32),
            pltpu.VMEM((CHUNK, EMBED + 2), jnp.float32),
            pltpu.SemaphoreType.DMA,
            pltpu.SemaphoreType.DMA,
        ],
    )
    out = run(tok, htb, pb, W)
    return out.reshape(BATCH, SEQ, EMBED + 2)


# speedup vs baseline: 2.2866x; 2.2866x over previous
import functools

import jax
import jax.numpy as jnp
from jax import lax
from jax.experimental import pallas as pl
from jax.experimental.pallas import tpu as pltpu, tpu_sc as plsc

WORD_COUNT = 1000000
NUM_BUCKETS = 100000
EMBED = 64
BATCH = 4096
SEQ = 200

NC = 2
NS = 16
L = 16
NW = NC * NS

N_TOK = BATCH * SEQ
TOK_PER_W = N_TOK // NW
CHUNK = 128
N_CHUNKS = TOK_PER_W // CHUNK
META_W = 16


def _sc_body(tok_hbm, tbl_hbm, w_hbm, out_hbm,
             tok_bufs, meta_bufs, idx0_bufs, idx1_bufs, p0_bufs, p1_bufs,
             w0_bufs, w1_bufs, out_bufs,
             tok_sems, meta_sems, w_sems, out_sems):
    wid = lax.axis_index("s") * NC + lax.axis_index("c")
    tile_base = wid * TOK_PER_W
    lane = lax.iota(jnp.int32, L)
    zeros = jnp.zeros((L,), jnp.int32)
    ones = jnp.full((L,), 1, jnp.int32)
    twos = jnp.full((L,), 2, jnp.int32)
    threes = jnp.full((L,), 3, jnp.int32)

    def tok_slice(q):
        return tok_hbm.at[pl.ds(tile_base + q * CHUNK, CHUNK)]

    def out_slice(q):
        return out_hbm.at[pl.ds(tile_base + q * CHUNK, CHUNK)]

    def issue_tok(q):
        pltpu.async_copy(tok_slice(q), tok_bufs.at[q % 4], tok_sems.at[q % 4])

    def wait_tok(q):
        pltpu.make_async_copy(
            tok_slice(q), tok_bufs.at[q % 4], tok_sems.at[q % 4]).wait()

    def issue_meta(q):
        pltpu.async_copy(
            tbl_hbm.at[tok_bufs.at[q % 4]], meta_bufs.at[q % 2],
            meta_sems.at[q % 2])

    def wait_meta(q):
        pltpu.make_async_copy(
            tbl_hbm.at[tok_bufs.at[q % 4]], meta_bufs.at[q % 2],
            meta_sems.at[q % 2]).wait()

    def issue_w(q):
        b = q % 2
        pltpu.async_copy(w_hbm.at[idx0_bufs.at[b]], w0_bufs.at[b],
                         w_sems.at[b])
        pltpu.async_copy(w_hbm.at[idx1_bufs.at[b]], w1_bufs.at[b],
                         w_sems.at[b])

    def wait_w(q):
        b = q % 2
        pltpu.make_async_copy(
            w_hbm.at[idx0_bufs.at[b]], w0_bufs.at[b], w_sems.at[b]).wait()
        pltpu.make_async_copy(
            w_hbm.at[idx1_bufs.at[b]], w1_bufs.at[b], w_sems.at[b]).wait()

    def issue_out(q):
        pltpu.async_copy(out_bufs.at[q % 2], out_slice(q), out_sems.at[q % 2])

    def wait_out(q):
        pltpu.make_async_copy(
            out_bufs.at[q % 2], out_slice(q), out_sems.at[q % 2]).wait()

    def meta_compute(q):
        b = q % 2
        tok_v = tok_bufs.at[q % 4]
        meta_v = meta_bufs.at[b]
        idx0_v = idx0_bufs.at[b]
        idx1_v = idx1_bufs.at[b]
        p0_v = p0_bufs.at[b]
        p1_v = p1_bufs.at[b]

        def grp(g, _):
            s = g * L
            rows = s + lane
            wv = tok_v[pl.ds(s, L)]
            nz = wv != 0
            b0 = plsc.load_gather(meta_v, [rows, zeros])
            b1 = plsc.load_gather(meta_v, [rows, ones])
            p0b = plsc.load_gather(meta_v, [rows, twos])
            p1b = plsc.load_gather(meta_v, [rows, threes])
            idx0_v[pl.ds(s, L)] = jnp.where(nz, b0, 0)
            idx1_v[pl.ds(s, L)] = jnp.where(nz, b1, 0)
            p0_v[pl.ds(s, L)] = plsc.bitcast(p0b, jnp.float32)
            p1_v[pl.ds(s, L)] = plsc.bitcast(p1b, jnp.float32)
            return 0

        lax.fori_loop(0, CHUNK // L, grp, 0)

    def out_compute(q):
        b = q % 2
        p0_v = p0_bufs.at[b]
        p1_v = p1_bufs.at[b]
        w0_v = w0_bufs.at[b]
        w1_v = w1_bufs.at[b]
        out_v = out_bufs.at[b]

        def tails(g, _):
            s = g * L
            rows = s + lane
            p0 = p0_v[pl.ds(s, L)]
            p1 = p1_v[pl.ds(s, L)]
            plsc.store_scatter(out_v, [rows, jnp.full((L,), EMBED, jnp.int32)], p0)
            plsc.store_scatter(out_v, [rows, jnp.full((L,), EMBED + 1, jnp.int32)], p1)
            return 0

        lax.fori_loop(0, CHUNK // L, tails, 0)

        def tok_body(i, _):
            p0 = plsc.load_gather(p0_v, [jnp.full((L,), i, jnp.int32)])
            p1 = plsc.load_gather(p1_v, [jnp.full((L,), i, jnp.int32)])
            for k in range(EMBED // L):
                a = w0_v[i, pl.ds(k * L, L)]
                b_ = w1_v[i, pl.ds(k * L, L)]
                out_v[i, pl.ds(k * L, L)] = a * p0 + b_ * p1
            return 0

        lax.fori_loop(0, CHUNK, tok_body, 0)

    pltpu.sync_copy(tok_slice(0), tok_bufs.at[0])
    pltpu.sync_copy(tok_slice(1), tok_bufs.at[1])
    issue_meta(0)
    issue_tok(2)
    wait_meta(0)
    meta_compute(0)
    issue_w(0)
    issue_meta(1)

    def iter_body(q, carry):
        wait_tok(q + 2)
        issue_meta(q + 2)

        @pl.when(q + 3 <= N_CHUNKS - 1)
        def _():
            issue_tok(q + 3)

        wait_meta(q + 1)
        meta_compute(q + 1)
        issue_w(q + 1)

        @pl.when(q >= 2)
        def _():
            wait_out(q - 2)

        wait_w(q)
        out_compute(q)
        issue_out(q)
        return carry

    lax.fori_loop(0, N_CHUNKS - 2, iter_body, 0)

    for q in (N_CHUNKS - 2, N_CHUNKS - 1):
        if q + 1 <= N_CHUNKS - 1:
            wait_meta(q + 1)
            meta_compute(q + 1)
            issue_w(q + 1)
        wait_out(q - 2)
        wait_w(q)
        out_compute(q)
        issue_out(q)
    wait_out(N_CHUNKS - 2)
    wait_out(N_CHUNKS - 1)


def kernel(input, hash_tables, p, W):
    tok = input.reshape(N_TOK)
    p_shift = jnp.roll(p, -3, axis=0)
    tbl = jnp.concatenate(
        [hash_tables,
         lax.bitcast_convert_type(p_shift, jnp.int32),
         jnp.zeros((WORD_COUNT, META_W - 4), jnp.int32)], axis=1)

    mesh = plsc.VectorSubcoreMesh(
        core_axis_name="c", subcore_axis_name="s",
        num_cores=NC, num_subcores=NS)
    run = pl.kernel(
        _sc_body,
        out_type=jax.ShapeDtypeStruct((N_TOK, EMBED + 2), jnp.float32),
        mesh=mesh,
        compiler_params=pltpu.CompilerParams(
            needs_layout_passes=False, use_tc_tiling_on_sc=False),
        scratch_types=[
            pltpu.VMEM((4, CHUNK), jnp.int32),
            pltpu.VMEM((2, CHUNK, META_W), jnp.int32),
            pltpu.VMEM((2, CHUNK), jnp.int32),
            pltpu.VMEM((2, CHUNK), jnp.int32),
            pltpu.VMEM((2, CHUNK), jnp.float32),
            pltpu.VMEM((2, CHUNK), jnp.float32),
            pltpu.VMEM((2, CHUNK, EMBED), jnp.float32),
            pltpu.VMEM((2, CHUNK, EMBED), jnp.float32),
            pltpu.VMEM((2, CHUNK, EMBED + 2), jnp.float32),
            pltpu.SemaphoreType.DMA((4,)),
            pltpu.SemaphoreType.DMA((2,)),
            pltpu.SemaphoreType.DMA((2,)),
            pltpu.SemaphoreType.DMA((2,)),
        ],
    )
    out = run(tok, tbl, W)
    return out.reshape(BATCH, SEQ, EMBED + 2)
